# baseline (speedup 1.0000x reference)
import functools

import jax
import jax.numpy as jnp
from jax import lax
from jax.experimental import pallas as pl
from jax.experimental.pallas import tpu as pltpu
from jax.experimental.pallas import tpu_sc as plsc

V = 24
D = 2048
B = 4096 * 20
NC, NS = 2, 16
NW = NC * NS
L = 16
ROWS_W = B // NW
SEC = 512
NSEC = ROWS_W // SEC
CH = SEC // L
NSEM = 2
DEPTH = 4

_mesh = plsc.VectorSubcoreMesh(core_axis_name="c", subcore_axis_name="s")


@functools.partial(
    pl.kernel,
    mesh=_mesh,
    out_type=jax.ShapeDtypeStruct((B, D), jnp.float32),
    scratch_types=[
        pltpu.VMEM((NSEC, SEC), jnp.int32),
        pltpu.VMEM_SHARED((V, D), jnp.float32),
        pltpu.SemaphoreType.DMA,
        pltpu.SemaphoreType.DMA,
    ],
)
def _codebook_gather(idx_hbm, table_hbm, out_hbm, idx_v, table_v, s0, s1):
    wid = lax.axis_index("s") * NC + lax.axis_index("c")
    base = wid * ROWS_W

    @pl.when(lax.axis_index("s") == 0)
    def _():
        pltpu.sync_copy(table_hbm, table_v)

    pltpu.sync_copy(idx_hbm.at[wid], idx_v)
    plsc.subcore_barrier()

    sems = (s0, s1)

    for sec in range(NSEC):
        sec_base = base + sec * SEC

        def issue_chunk(g, sem):
            vec = idx_v[sec, pl.ds(g * L, L)]
            for j in range(L):
                sidx = vec[j]
                pltpu.make_async_copy(
                    table_v.at[pl.ds(sidx, 1)],
                    out_hbm.at[pl.ds(sec_base + g * L + j, 1)],
                    sem).start()

        def wait_chunk(g, sem):
            pltpu.make_async_copy(
                table_v.at[pl.ds(0, L)],
                out_hbm.at[pl.ds(sec_base + g * L, L)],
                sem).wait()

        def body(i, carry):
            for p in range(NSEM):
                g = NSEM * i + p

                @pl.when(g >= DEPTH)
                def _():
                    wait_chunk(g - DEPTH, sems[p])

                issue_chunk(g, sems[p])
            return carry

        lax.fori_loop(0, CH // NSEM, body, None)

        for d in range(DEPTH):
            g_last = CH - DEPTH + d
            wait_chunk(g_last, sems[g_last % NSEM])


def kernel(x, table):
    idx = x.astype(jnp.int32).reshape(NW, NSEC, SEC)
    out = _codebook_gather(idx, table)
    return out.reshape(x.shape[0], x.shape[1], D)


# speedup vs baseline: 1.0811x; 1.0811x over previous
import functools

import jax
import jax.numpy as jnp
from jax import lax
from jax.experimental import pallas as pl
from jax.experimental.pallas import tpu as pltpu
from jax.experimental.pallas import tpu_sc as plsc

V = 24
D = 2048
B = 4096 * 20
NC, NS = 2, 16
NW = NC * NS
L = 16
ROWS_W = B // NW
SEC = 512
NSEC = ROWS_W // SEC
CH = SEC // L
NSEM = 2
DEPTH = 4

_mesh = plsc.VectorSubcoreMesh(core_axis_name="c", subcore_axis_name="s")


@functools.partial(
    pl.kernel,
    mesh=_mesh,
    out_type=jax.ShapeDtypeStruct((B, D), jnp.float32),
    scratch_types=[
        pltpu.VMEM((NSEC, SEC), jnp.int32),
        pltpu.VMEM((V, D), jnp.float32),
        pltpu.SemaphoreType.DMA,
        pltpu.SemaphoreType.DMA,
    ],
)
def _codebook_gather(idx_hbm, table_hbm, out_hbm, idx_v, table_v, s0, s1):
    wid = lax.axis_index("s") * NC + lax.axis_index("c")
    base = wid * ROWS_W
    pltpu.sync_copy(table_hbm, table_v)
    pltpu.sync_copy(idx_hbm.at[wid], idx_v)

    sems = (s0, s1)

    for sec in range(NSEC):
        sec_base = base + sec * SEC

        def issue_chunk(g, sem):
            vec = idx_v[sec, pl.ds(g * L, L)]
            for j in range(L):
                sidx = vec[j]
                pltpu.make_async_copy(
                    table_v.at[pl.ds(sidx, 1)],
                    out_hbm.at[pl.ds(sec_base + g * L + j, 1)],
                    sem).start()

        def wait_chunk(g, sem):
            pltpu.make_async_copy(
                table_v.at[pl.ds(0, L)],
                out_hbm.at[pl.ds(sec_base + g * L, L)],
                sem).wait()

        def body(i, carry):
            for p in range(NSEM):
                g = NSEM * i + p

                @pl.when(g >= DEPTH)
                def _():
                    wait_chunk(g - DEPTH, sems[p])

                issue_chunk(g, sems[p])
            return carry

        lax.fori_loop(0, CH // NSEM, body, None)

        for d in range(DEPTH):
            g_last = CH - DEPTH + d
            wait_chunk(g_last, sems[g_last % NSEM])


def kernel(x, table):
    idx = x.astype(jnp.int32).reshape(NW, NSEC, SEC)
    out = _codebook_gather(idx, table)
    return out.reshape(x.shape[0], x.shape[1], D)
